# Initial kernel scaffold; baseline (speedup 1.0000x reference)
import jax
import jax.numpy as jnp
from jax.experimental import pallas as pl


def kernel(x, edge_index, edge_attr, W_pre, b_pre, W1, b1, W2, b2, W_head, b_head):
    raise NotImplementedError("write your pallas kernel here")


# speedup vs baseline: 3.0236x; 3.0236x over previous
import functools

import jax
import jax.numpy as jnp
from jax import lax
from jax.experimental import pallas as pl
from jax.experimental.pallas import tpu as pltpu
from jax.experimental.pallas import tpu_sc as plsc


_NC = 2
_NS = 16
_LANES = 16


@functools.lru_cache(maxsize=None)
def _make_sc_msgpass(N, E, D):
    NW = _NC * _NS
    assert E % NW == 0
    epw = E // NW
    K = 80
    assert epw % K == 0
    nchunk = epw // K
    blk = _NS * 128
    N_pad = ((N + blk - 1) // blk) * blk
    rows_per_tile = N_pad // _NS
    ZR = 128
    assert rows_per_tile % ZR == 0
    nzero = rows_per_tile // ZR
    cols = D // _LANES

    mesh = plsc.VectorSubcoreMesh(core_axis_name="c", subcore_axis_name="s",
                                  num_cores=_NC, num_subcores=_NS)

    @functools.partial(
        pl.kernel,
        out_type=jax.ShapeDtypeStruct((_NC, N_pad, D), jnp.float32),
        mesh=mesh,
        scratch_types=dict(
            src_v=pltpu.VMEM((K,), jnp.int32),
            dst_v=pltpu.VMEM((K,), jnp.int32),
            ea_v=pltpu.VMEM((K, D), jnp.float32),
            hg_v=pltpu.VMEM((K, D), jnp.float32),
            zb_v=pltpu.VMEM((ZR, D), jnp.float32),
            agg_sh=pltpu.VMEM_SHARED((N_pad, D), jnp.float32),
            sem=pltpu.SemaphoreType.DMA,
        ),
    )
    def sc_msgpass(h_hbm, src_hbm, dst_hbm, ea_hbm, out_hbm,
                   src_v, dst_v, ea_v, hg_v, zb_v, agg_sh, sem):
        c = lax.axis_index("c")
        s = lax.axis_index("s")
        wid = c * _NS + s

        def zrow(i, carry):
            for j in range(cols):
                zb_v[i, pl.ds(j * _LANES, _LANES)] = jnp.zeros(
                    (_LANES,), jnp.float32)
            return carry
        lax.fori_loop(0, ZR, zrow, 0)
        row0 = s * rows_per_tile
        for t in range(nzero):
            pltpu.sync_copy(zb_v, agg_sh.at[pl.ds(row0 + t * ZR, ZR)])
        plsc.subcore_barrier()

        ebase = wid * epw

        def chunk(k, carry):
            b = ebase + k * K
            pltpu.sync_copy(src_hbm.at[pl.ds(b, K)], src_v)
            pltpu.sync_copy(dst_hbm.at[pl.ds(b, K)], dst_v)
            pltpu.sync_copy(ea_hbm.at[pl.ds(b, K)], ea_v)
            pltpu.async_copy(h_hbm.at[src_v], hg_v, sem).wait()

            def mrow(i, carry2):
                for j in range(cols):
                    sl = pl.ds(j * _LANES, _LANES)
                    ea_v[i, sl] = jnp.maximum(hg_v[i, sl] + ea_v[i, sl], 0.0)
                return carry2
            lax.fori_loop(0, K, mrow, 0)

            pltpu.sync_copy(ea_v, agg_sh.at[dst_v], add=True)
            return carry
        lax.fori_loop(0, nchunk, chunk, 0)
        plsc.subcore_barrier()

        pltpu.sync_copy(agg_sh.at[pl.ds(row0, rows_per_tile)],
                        out_hbm.at[c, pl.ds(row0, rows_per_tile)])

    return sc_msgpass


_BM = 2000


def _pre_body(x_ref, w_ref, b_ref, o_ref):
    acc = jnp.dot(x_ref[...], w_ref[...], preferred_element_type=jnp.float32)
    o_ref[...] = jnp.maximum(acc + b_ref[...], 0.0)


def _mlp_body(h_ref, a_ref, w1_ref, b1_ref, w2_ref, b2_ref, o_ref):
    h = h_ref[...]
    z = h + a_ref[0] + a_ref[1]
    z1 = jnp.maximum(
        jnp.dot(z, w1_ref[...], preferred_element_type=jnp.float32)
        + b1_ref[...], 0.0)
    z2 = (jnp.dot(z1, w2_ref[...], preferred_element_type=jnp.float32)
          + b2_ref[...])
    o_ref[...] = h + jnp.maximum(z2, 0.0)


def _head_body(h_ref, w_ref, b_ref, o_ref):
    o_ref[...] = (jnp.dot(h_ref[...], w_ref[...],
                          preferred_element_type=jnp.float32) + b_ref[...])


def _row_block(bm, d):
    return pl.BlockSpec((bm, d), lambda i: (i, 0))


def _full_block(shape):
    return pl.BlockSpec(shape, lambda i: tuple(0 for _ in shape))


def _tc_pre(x, W, b):
    N, D = x.shape
    return pl.pallas_call(
        _pre_body,
        grid=(N // _BM,),
        in_specs=[_row_block(_BM, D), _full_block(W.shape),
                  _full_block(b.shape)],
        out_specs=_row_block(_BM, W.shape[1]),
        out_shape=jax.ShapeDtypeStruct((N, W.shape[1]), jnp.float32),
    )(x, W, b)


def _tc_mlp(h, agg2, W1, b1, W2, b2):
    N, D = h.shape
    return pl.pallas_call(
        _mlp_body,
        grid=(N // _BM,),
        in_specs=[
            _row_block(_BM, D),
            pl.BlockSpec((_NC, _BM, D), lambda i: (0, i, 0)),
            _full_block(W1.shape), _full_block(b1.shape),
            _full_block(W2.shape), _full_block(b2.shape),
        ],
        out_specs=_row_block(_BM, D),
        out_shape=jax.ShapeDtypeStruct((N, D), jnp.float32),
    )(h, agg2, W1, b1, W2, b2)


def _tc_head(h, W, b):
    N, D = h.shape
    OUT = W.shape[1]
    return pl.pallas_call(
        _head_body,
        grid=(N // _BM,),
        in_specs=[_row_block(_BM, D), _full_block(W.shape),
                  _full_block(b.shape)],
        out_specs=_row_block(_BM, OUT),
        out_shape=jax.ShapeDtypeStruct((N, OUT), jnp.float32),
    )(h, W, b)


def kernel(x, edge_index, edge_attr, W_pre, b_pre, W1, b1, W2, b2,
           W_head, b_head):
    N, D = x.shape
    E = edge_attr.shape[0]
    L = W1.shape[0]
    src = edge_index[0]
    dst = edge_index[1]

    sc_msgpass = _make_sc_msgpass(N, E, D)

    h = _tc_pre(x, W_pre, b_pre.reshape(1, D))
    for l in range(L):
        agg2 = sc_msgpass(h, src, dst, edge_attr)
        h = _tc_mlp(h, agg2, W1[l], b1[l].reshape(1, D),
                    W2[l], b2[l].reshape(1, D))
    return _tc_head(h, W_head, b_head.reshape(1, -1))
